# baseline (speedup 1.0000x reference)
import functools

import jax
import jax.numpy as jnp
from jax import lax
from jax.experimental import pallas as pl
from jax.experimental.pallas import tpu as pltpu
from jax.experimental.pallas import tpu_sc as plsc

DIM = 64
NBUF = 4


def _emb_call(idx, weight, b, s):
    num_rows = b * s
    info = plsc.get_sparse_core_info()
    nc, ns = info.num_cores, info.num_subcores
    nw = nc * ns
    rows_per_w = num_rows // nw
    planes_per_w = rows_per_w // s
    n_outer = planes_per_w // NBUF

    mesh = plsc.VectorSubcoreMesh(core_axis_name="c", subcore_axis_name="s")

    @functools.partial(
        pl.kernel,
        mesh=mesh,
        out_type=jax.ShapeDtypeStruct((b, s, DIM), jnp.float32),
        scratch_types=[
            pltpu.VMEM((rows_per_w,), jnp.int32),
            pltpu.VMEM((NBUF, s, DIM), jnp.float32),
            [pltpu.SemaphoreType.DMA] * NBUF,
            [pltpu.SemaphoreType.DMA] * NBUF,
        ],
        compiler_params=pltpu.CompilerParams(use_tc_tiling_on_sc=False),
    )
    def emb(idx_hbm, table_hbm, out_hbm, idx_v, rows_v, gsems, ssems):
        wid = lax.axis_index("s") * nc + lax.axis_index("c")
        base = wid * rows_per_w
        b0 = wid * planes_per_w
        pltpu.sync_copy(idx_hbm.at[pl.ds(base, rows_per_w)], idx_v)

        def outer(g, carry):
            for k in range(NBUF):
                i = g * NBUF + k

                @pl.when(g > 0)
                def _wait_store():
                    pltpu.make_async_copy(
                        rows_v.at[k], out_hbm.at[b0 + i - NBUF], ssems[k]
                    ).wait()

                pltpu.async_copy(
                    table_hbm.at[idx_v.at[pl.ds(i * s, s)]],
                    rows_v.at[k],
                    gsems[k],
                )
            for k in range(NBUF):
                i = g * NBUF + k
                pltpu.make_async_copy(
                    table_hbm.at[idx_v.at[pl.ds(i * s, s)]],
                    rows_v.at[k],
                    gsems[k],
                ).wait()
                pltpu.async_copy(rows_v.at[k], out_hbm.at[b0 + i], ssems[k])
            return carry

        lax.fori_loop(0, n_outer, outer, 0)

        for k in range(NBUF):
            i = (n_outer - 1) * NBUF + k
            pltpu.make_async_copy(
                rows_v.at[k], out_hbm.at[b0 + i], ssems[k]
            ).wait()

    return emb(idx, weight)


def kernel(tokens, weight):
    b, s = tokens.shape
    idx = tokens.reshape(b * s).astype(jnp.int32)
    return _emb_call(idx, weight, b, s)
